# Initial kernel scaffold; baseline (speedup 1.0000x reference)
import jax
import jax.numpy as jnp
from jax.experimental import pallas as pl


def kernel(feature_map, switches, output_size):
    raise NotImplementedError("write your pallas kernel here")


# speedup vs baseline: 38.7040x; 38.7040x over previous
import functools

import jax
import jax.numpy as jnp
from jax import lax
from jax.experimental import pallas as pl
from jax.experimental.pallas import tpu as pltpu
from jax.experimental.pallas import tpu_sc as plsc

L = 16


@functools.partial(jax.jit, static_argnums=(2, 3))
def _sc_unpool(fm_flat, sw_flat, n_rows, wp):
    info = plsc.get_sparse_core_info()
    nc, ns = info.num_cores, info.num_subcores
    nw = nc * ns

    hb = 8
    rows_per_w = n_rows // nw
    n_chunks = rows_per_w // hb
    in_chunk = hb * wp
    out_chunk = 2 * hb * 2 * wp
    w2 = 2 * wp
    nvec = in_chunk // L

    mesh = plsc.VectorSubcoreMesh(core_axis_name="c", subcore_axis_name="s")

    @functools.partial(
        pl.kernel,
        out_type=jax.ShapeDtypeStruct((4 * n_rows * wp,), jnp.float32),
        mesh=mesh,
        compiler_params=pltpu.CompilerParams(needs_layout_passes=False),
        scratch_types=[
            pltpu.VMEM((in_chunk,), jnp.float32),
            pltpu.VMEM((in_chunk,), jnp.int32),
            pltpu.VMEM((out_chunk,), jnp.float32),
        ],
    )
    def k(fm_hbm, sw_hbm, out_hbm, fm_v, sw_v, out_v):
        wid = lax.axis_index("s") * nc + lax.axis_index("c")
        iota = lax.iota(jnp.int32, L)
        two_iota = iota * 2
        zeros = jnp.zeros((L,), jnp.float32)

        def body(t, _):
            ibase = (wid * n_chunks + t) * in_chunk
            obase = (wid * n_chunks + t) * out_chunk
            pltpu.sync_copy(fm_hbm.at[pl.ds(ibase, in_chunk)], fm_v)
            pltpu.sync_copy(sw_hbm.at[pl.ds(ibase, in_chunk)], sw_v)
            for j in range(out_chunk // L):
                out_v[pl.ds(j * L, L)] = zeros
            for j in range(nvec):
                row_l = (j * L) // wp
                col0 = (j * L) % wp
                base = row_l * 2 * w2 + 2 * col0
                s = sw_v[pl.ds(j * L, L)]
                v = fm_v[pl.ds(j * L, L)]
                idx = (base + two_iota) + jnp.where(s >= 2, w2, 0) + (s & 1)
                plsc.store_scatter(out_v, [idx], v)
            pltpu.sync_copy(out_v, out_hbm.at[pl.ds(obase, out_chunk)])
            return ()

        lax.fori_loop(0, n_chunks, body, (), unroll=False)

    return k(fm_flat, sw_flat)


def kernel(feature_map, switches, output_size):
    B, C, Hp, Wp = feature_map.shape
    n_rows = B * C * Hp
    out_flat = _sc_unpool(
        feature_map.reshape(-1), switches.reshape(-1), n_rows, Wp
    )
    return out_flat.reshape(B, C, 2 * Hp, 2 * Wp)


# speedup vs baseline: 62.9589x; 1.6267x over previous
import functools

import jax
import jax.numpy as jnp
from jax import lax
from jax.experimental import pallas as pl
from jax.experimental.pallas import tpu as pltpu
from jax.experimental.pallas import tpu_sc as plsc

L = 16
HB = 16


@functools.partial(jax.jit, static_argnums=(2, 3))
def _sc_unpool(fm_flat, sw_flat, n_rows, wp):
    info = plsc.get_sparse_core_info()
    nc, ns = info.num_cores, info.num_subcores
    nw = nc * ns

    hb = HB
    rows_per_w = n_rows // nw
    n_chunks = rows_per_w // hb
    in_chunk = hb * wp
    w2 = 2 * wp
    out_chunk = 2 * hb * w2
    ngrp = wp // L

    mesh = plsc.VectorSubcoreMesh(core_axis_name="c", subcore_axis_name="s")

    @functools.partial(
        pl.kernel,
        out_type=jax.ShapeDtypeStruct((4 * n_rows * wp,), jnp.float32),
        mesh=mesh,
        compiler_params=pltpu.CompilerParams(needs_layout_passes=False),
        scratch_types=[
            pltpu.VMEM((in_chunk,), jnp.float32),
            pltpu.VMEM((in_chunk,), jnp.float32),
            pltpu.VMEM((in_chunk,), jnp.int32),
            pltpu.VMEM((in_chunk,), jnp.int32),
            pltpu.VMEM((out_chunk,), jnp.float32),
            pltpu.VMEM((out_chunk,), jnp.float32),
            pltpu.SemaphoreType.DMA,
            pltpu.SemaphoreType.DMA,
            pltpu.SemaphoreType.DMA,
            pltpu.SemaphoreType.DMA,
            pltpu.SemaphoreType.DMA,
            pltpu.SemaphoreType.DMA,
        ],
    )
    def k(fm_hbm, sw_hbm, out_hbm,
          fm0, fm1, sw0, sw1, ov0, ov1,
          sf0, sf1, ss0, ss1, so0, so1):
        wid = lax.axis_index("s") * nc + lax.axis_index("c")
        base0 = wid * n_chunks
        fm_b = (fm0, fm1)
        sw_b = (sw0, sw1)
        ov_b = (ov0, ov1)
        sf = (sf0, sf1)
        ss = (ss0, ss1)
        so = (so0, so1)

        iota = lax.iota(jnp.int32, L)
        two_iota = iota * 2
        zeros = jnp.zeros((L,), jnp.float32)

        def start_in(t, b):
            ibase = (base0 + t) * in_chunk
            pltpu.async_copy(fm_hbm.at[pl.ds(ibase, in_chunk)], fm_b[b], sf[b])
            pltpu.async_copy(sw_hbm.at[pl.ds(ibase, in_chunk)], sw_b[b], ss[b])

        def wait_in(t, b):
            ibase = (base0 + t) * in_chunk
            pltpu.make_async_copy(
                fm_hbm.at[pl.ds(ibase, in_chunk)], fm_b[b], sf[b]).wait()
            pltpu.make_async_copy(
                sw_hbm.at[pl.ds(ibase, in_chunk)], sw_b[b], ss[b]).wait()

        def start_out(t, b):
            obase = (base0 + t) * out_chunk
            pltpu.async_copy(ov_b[b], out_hbm.at[pl.ds(obase, out_chunk)], so[b])

        def wait_out(t, b):
            obase = (base0 + t) * out_chunk
            pltpu.make_async_copy(
                ov_b[b], out_hbm.at[pl.ds(obase, out_chunk)], so[b]).wait()

        def compute(b):
            fm_v, sw_v, out_v = fm_b[b], sw_b[b], ov_b[b]

            def row_body(r, _):
                zb = r * 2 * w2
                for g in range(4 * ngrp):
                    out_v[pl.ds(zb + g * L, L)] = zeros
                ib = r * wp
                for g in range(ngrp):
                    s = sw_v[pl.ds(ib + g * L, L)]
                    v = fm_v[pl.ds(ib + g * L, L)]
                    idx = ((zb + g * 2 * L) + two_iota
                           + jnp.where(s >= 2, w2, 0) + (s & 1))
                    plsc.store_scatter(out_v, [idx], v)
                return ()

            lax.fori_loop(0, hb, row_body, (), unroll=False)

        start_in(0, 0)
        start_in(1, 1)
        for b in range(2):
            wait_in(b, b)
            compute(b)
            start_out(b, b)
            start_in(b + 2, b)

        def body(i, _):
            for b in range(2):
                t = 2 * i + b
                wait_in(t, b)
                wait_out(t - 2, b)
                compute(b)
                start_out(t, b)

                @pl.when(t + 2 < n_chunks)
                def _():
                    start_in(t + 2, b)

            return ()

        lax.fori_loop(1, n_chunks // 2, body, (), unroll=False)

        wait_out(n_chunks - 2, 0)
        wait_out(n_chunks - 1, 1)

    return k(fm_flat, sw_flat)


def kernel(feature_map, switches, output_size):
    B, C, Hp, Wp = feature_map.shape
    n_rows = B * C * Hp
    out_flat = _sc_unpool(
        feature_map.reshape(-1), switches.reshape(-1), n_rows, Wp
    )
    return out_flat.reshape(B, C, 2 * Hp, 2 * Wp)


# speedup vs baseline: 99.0579x; 1.5734x over previous
import functools

import jax
import jax.numpy as jnp
from jax import lax
from jax.experimental import pallas as pl
from jax.experimental.pallas import tpu as pltpu
from jax.experimental.pallas import tpu_sc as plsc

L = 16
HB = 8


@functools.partial(jax.jit, static_argnums=(2,))
def _sc_unpool(fm, sw, wp):
    n_rows = fm.shape[0]
    info = plsc.get_sparse_core_info()
    nc, ns = info.num_cores, info.num_subcores
    nw = nc * ns

    hb = HB
    rows_per_w = n_rows // nw
    n_chunks = rows_per_w // hb
    w2 = 2 * wp
    ngrp = wp // L

    mesh = plsc.VectorSubcoreMesh(core_axis_name="c", subcore_axis_name="s")

    @functools.partial(
        pl.kernel,
        out_type=jax.ShapeDtypeStruct((2 * n_rows, w2), jnp.float32),
        mesh=mesh,
        compiler_params=pltpu.CompilerParams(needs_layout_passes=False),
        scratch_types=[
            pltpu.VMEM((hb, wp), jnp.float32),
            pltpu.VMEM((hb, wp), jnp.float32),
            pltpu.VMEM((hb, wp), jnp.int32),
            pltpu.VMEM((hb, wp), jnp.int32),
            pltpu.VMEM((2 * hb, w2), jnp.float32),
            pltpu.VMEM((2 * hb, w2), jnp.float32),
            pltpu.SemaphoreType.DMA,
            pltpu.SemaphoreType.DMA,
            pltpu.SemaphoreType.DMA,
            pltpu.SemaphoreType.DMA,
            pltpu.SemaphoreType.DMA,
            pltpu.SemaphoreType.DMA,
        ],
    )
    def k(fm_hbm, sw_hbm, out_hbm,
          fm0, fm1, sw0, sw1, ov0, ov1,
          sf0, sf1, ss0, ss1, so0, so1):
        wid = lax.axis_index("s") * nc + lax.axis_index("c")
        row0 = wid * rows_per_w
        fm_b = (fm0, fm1)
        sw_b = (sw0, sw1)
        ov_b = (ov0, ov1)
        sf = (sf0, sf1)
        ss = (ss0, ss1)
        so = (so0, so1)

        iota = lax.iota(jnp.int32, L)
        two_iota = iota * 2
        zeros = jnp.zeros((L,), jnp.float32)

        def start_in(t, b):
            r = row0 + t * hb
            pltpu.async_copy(fm_hbm.at[pl.ds(r, hb), :], fm_b[b], sf[b])
            pltpu.async_copy(sw_hbm.at[pl.ds(r, hb), :], sw_b[b], ss[b])

        def wait_in(t, b):
            r = row0 + t * hb
            pltpu.make_async_copy(
                fm_hbm.at[pl.ds(r, hb), :], fm_b[b], sf[b]).wait()
            pltpu.make_async_copy(
                sw_hbm.at[pl.ds(r, hb), :], sw_b[b], ss[b]).wait()

        def start_out(t, b):
            r = 2 * (row0 + t * hb)
            pltpu.async_copy(ov_b[b], out_hbm.at[pl.ds(r, 2 * hb), :], so[b])

        def wait_out(t, b):
            r = 2 * (row0 + t * hb)
            pltpu.make_async_copy(
                ov_b[b], out_hbm.at[pl.ds(r, 2 * hb), :], so[b]).wait()

        def compute(b):
            fm_v, sw_v, out_v = fm_b[b], sw_b[b], ov_b[b]
            for r in range(hb):
                for g in range(4 * ngrp):
                    out_v[2 * r + g // (2 * ngrp),
                          pl.ds((g % (2 * ngrp)) * L, L)] = zeros
                for g in range(ngrp):
                    s = sw_v[r, pl.ds(g * L, L)]
                    v = fm_v[r, pl.ds(g * L, L)]
                    idx_r = 2 * r + jnp.where(s >= 2, 1, 0)
                    idx_c = (g * 2 * L) + two_iota + (s & 1)
                    plsc.store_scatter(out_v, [idx_r, idx_c], v)

        start_in(0, 0)
        start_in(1, 1)
        for b in range(2):
            wait_in(b, b)
            compute(b)
            start_out(b, b)
            start_in(b + 2, b)

        def body(i, _):
            for b in range(2):
                t = 2 * i + b
                wait_in(t, b)
                wait_out(t - 2, b)
                compute(b)
                start_out(t, b)

                @pl.when(t + 2 < n_chunks)
                def _():
                    start_in(t + 2, b)

            return ()

        lax.fori_loop(1, n_chunks // 2, body, (), unroll=False)

        wait_out(n_chunks - 2, 0)
        wait_out(n_chunks - 1, 1)

    return k(fm, sw)


def kernel(feature_map, switches, output_size):
    B, C, Hp, Wp = feature_map.shape
    out2 = _sc_unpool(
        feature_map.reshape(B * C * Hp, Wp),
        switches.reshape(B * C * Hp, Wp),
        Wp,
    )
    return out2.reshape(B, C, 2 * Hp, 2 * Wp)
